# baseline (speedup 1.0000x reference)
import functools

import jax
import jax.numpy as jnp
from jax import lax
from jax.experimental import pallas as pl
from jax.experimental.pallas import tpu as pltpu
from jax.experimental.pallas import tpu_sc as plsc

N = 10000
DIN = 256
DH = 512
LANE = 128
NC, NS = 2, 16
R = N
ROWS_PT = 640
ROWS_LAST = R - (NS - 1) * ROWS_PT
E_PAD = 163840
PAD_CNT = E_PAD - 160000


def _sc_mesh():
    return plsc.VectorSubcoreMesh(
        core_axis_name="c", subcore_axis_name="s", num_cores=NC, num_subcores=NS
    )


def _make_segsum(C):
    cps = C // NC
    nj = E_PAD // NS // LANE
    nh = nj // 2
    q_depth = 2

    def body(hc, edges, zeros, out, src_v, dst_v, buf, fbuf, acc, gsem):
        cid = lax.axis_index("c")
        sid = lax.axis_index("s")
        for k in range(cps):
            chunk = cid * cps + k

            @pl.when(sid < NS - 1)
            def _():
                pltpu.sync_copy(zeros, acc.at[pl.ds(sid * ROWS_PT, ROWS_PT)])

            @pl.when(sid == NS - 1)
            def _():
                pltpu.sync_copy(zeros.at[pl.ds(0, ROWS_LAST)],
                                acc.at[pl.ds(sid * ROWS_PT, ROWS_LAST)])

            plsc.subcore_barrier()
            for half in range(2):
                base = sid * nj + half * nh
                pltpu.sync_copy(edges.at[0, pl.ds(base, nh)], src_v)
                pltpu.sync_copy(edges.at[1, pl.ds(base, nh)], dst_v)
                for q in range(q_depth):
                    pltpu.async_copy(hc.at[chunk].at[src_v.at[q]], buf.at[q],
                                     gsem.at[q])

                def outer(jo, carry):
                    for q in range(q_depth):
                        j = jo * q_depth + q
                        pltpu.make_async_copy(
                            hc.at[chunk].at[src_v.at[q]], buf.at[q], gsem.at[q]
                        ).wait()

                        @plsc.parallel_loop(0, LANE, unroll=8)
                        def _(r):
                            for g in range(4):
                                v = buf[q, r, pl.ds(16 * g, 16)]
                                vb = plsc.bitcast(v, jnp.bfloat16)
                                a, b = plsc.unpack(
                                    vb, format=plsc.PackFormat.INTERLEAVED)
                                fbuf[r, pl.ds(16 * g, 16)] = a
                                fbuf[r, pl.ds(64 + 16 * g, 16)] = b

                        @pl.when(jo < nh // q_depth - 1)
                        def _():
                            pltpu.async_copy(
                                hc.at[chunk].at[src_v.at[j + q_depth]],
                                buf.at[q], gsem.at[q],
                            )
                        pltpu.sync_copy(fbuf, acc.at[dst_v.at[j]], add=True)
                    return carry

                lax.fori_loop(0, nh // q_depth, outer, 0)
            plsc.subcore_barrier()

            @pl.when(sid < NS - 1)
            def _():
                pltpu.sync_copy(
                    acc.at[pl.ds(sid * ROWS_PT, ROWS_PT)],
                    out.at[chunk, pl.ds(sid * ROWS_PT, ROWS_PT)],
                )

            @pl.when(sid == NS - 1)
            def _():
                pltpu.sync_copy(
                    acc.at[pl.ds(sid * ROWS_PT, ROWS_LAST)],
                    out.at[chunk, pl.ds(sid * ROWS_PT, ROWS_LAST)],
                )

            plsc.subcore_barrier()

    return pl.kernel(
        body,
        out_type=jax.ShapeDtypeStruct((C, R, LANE), jnp.float32),
        mesh=_sc_mesh(),
        compiler_params=pltpu.CompilerParams(
            needs_layout_passes=False, use_tc_tiling_on_sc=False),
        scratch_types=[
            pltpu.VMEM((nh, LANE), jnp.int32),
            pltpu.VMEM((nh, LANE), jnp.int32),
            pltpu.VMEM((q_depth, LANE, 64), jnp.int32),
            pltpu.VMEM((LANE, LANE), jnp.float32),
            pltpu.VMEM_SHARED((R, LANE), jnp.float32),
            pltpu.SemaphoreType.DMA((q_depth,)),
        ],
    )


def _deg_body(edges, zeros, ones, out, dst_v, buf, acc):
    njd = E_PAD // (NC * NS) // LANE
    cid = lax.axis_index("c")
    sid = lax.axis_index("s")
    wid = cid * NS + sid
    pltpu.sync_copy(edges.at[1, pl.ds(wid * njd, njd)], dst_v)
    pltpu.sync_copy(ones, buf)

    @pl.when(sid < NS - 1)
    def _():
        pltpu.sync_copy(zeros, acc.at[pl.ds(sid * ROWS_PT, ROWS_PT)])

    @pl.when(sid == NS - 1)
    def _():
        pltpu.sync_copy(zeros.at[pl.ds(0, ROWS_LAST)],
                        acc.at[pl.ds(sid * ROWS_PT, ROWS_LAST)])

    plsc.subcore_barrier()

    def step(j, carry):
        pltpu.sync_copy(buf, acc.at[dst_v.at[j]], add=True)
        return carry

    lax.fori_loop(0, njd, step, 0)
    plsc.subcore_barrier()

    @pl.when(sid < NS - 1)
    def _():
        pltpu.sync_copy(
            acc.at[pl.ds(sid * ROWS_PT, ROWS_PT)],
            out.at[cid, pl.ds(sid * ROWS_PT, ROWS_PT)],
        )

    @pl.when(sid == NS - 1)
    def _():
        pltpu.sync_copy(
            acc.at[pl.ds(sid * ROWS_PT, ROWS_LAST)],
            out.at[cid, pl.ds(sid * ROWS_PT, ROWS_LAST)],
        )


_deg_kernel = None


def _get_deg_kernel():
    global _deg_kernel
    if _deg_kernel is None:
        _deg_kernel = pl.kernel(
            _deg_body,
            out_type=jax.ShapeDtypeStruct((NC, R, LANE), jnp.float32),
            mesh=_sc_mesh(),
            scratch_types=[
                pltpu.VMEM((E_PAD // (NC * NS) // LANE, LANE), jnp.int32),
                pltpu.VMEM((LANE, LANE), jnp.float32),
                pltpu.VMEM_SHARED((R, LANE), jnp.float32),
            ],
        )
    return _deg_kernel


def _pack_chunk(X):
    lo = lax.bitcast_convert_type(
        X[:, :64].astype(jnp.bfloat16), jnp.uint16).astype(jnp.uint32)
    hi = lax.bitcast_convert_type(
        X[:, 64:].astype(jnp.bfloat16), jnp.uint16).astype(jnp.uint32)
    return lax.bitcast_convert_type(lo | (hi << 16), jnp.int32)


def _chunkify_body(x_ref, out_ref):
    for c in range(DIN // LANE):
        out_ref[c] = _pack_chunk(x_ref[:, c * LANE:(c + 1) * LANE])


def _chunkify(x):
    c = DIN // LANE
    return pl.pallas_call(
        _chunkify_body,
        grid=(N // BR,),
        in_specs=[pl.BlockSpec((BR, DIN), lambda i: (i, 0))],
        out_specs=pl.BlockSpec((c, BR, 64), lambda i: (0, i, 0)),
        out_shape=jax.ShapeDtypeStruct((c, N, 64), jnp.int32),
    )(x)


BR = 400


def _layer_body(mode, C_in, h_ref, S_ref, deg_ref, sc_ref, W1_ref, b1_ref,
                W2_ref, b2_ref, *rest):
    if mode == 0:
        Wres_ref, out_ref, hc_ref = rest
    elif mode == 1:
        out_ref, hc_ref = rest
    else:
        (out_ref,) = rest
    h = h_ref[...]
    corr = jnp.where(
        (lax.broadcasted_iota(jnp.int32, (h.shape[0], 1), 0) == 0)
        & (pl.program_id(0) == 0),
        jnp.float32(PAD_CNT), jnp.float32(0.0),
    )
    deg = jnp.maximum(deg_ref[0][:, :1] + deg_ref[1][:, :1] - corr, 1.0)
    S = jnp.concatenate([S_ref[c] for c in range(C_in)], axis=1)
    pre = sc_ref[...] * h + (S - corr * h) / deg
    mid = jnp.maximum(
        jnp.dot(pre, W1_ref[...], preferred_element_type=jnp.float32) + b1_ref[...],
        0.0,
    )
    out = jnp.dot(mid, W2_ref[...], preferred_element_type=jnp.float32) + b2_ref[...]
    if mode == 0:
        out = jnp.maximum(out, 0.0) + jnp.dot(
            h, Wres_ref[...], preferred_element_type=jnp.float32
        )
    elif mode == 1:
        out = jnp.maximum(out, 0.0) + h
    out_ref[...] = out
    if mode < 2:
        for c in range(DH // LANE):
            hc_ref[c] = _pack_chunk(out[:, c * LANE:(c + 1) * LANE])


def _make_layer(K, C_in, mode):
    in_specs = [
        pl.BlockSpec((BR, K), lambda i: (i, 0)),
        pl.BlockSpec((C_in, BR, LANE), lambda i: (0, i, 0)),
        pl.BlockSpec((NC, BR, LANE), lambda i: (0, i, 0)),
        pl.BlockSpec((1, 1), lambda i: (0, 0)),
        pl.BlockSpec((K, DH), lambda i: (0, 0)),
        pl.BlockSpec((1, DH), lambda i: (0, 0)),
        pl.BlockSpec((DH, DH), lambda i: (0, 0)),
        pl.BlockSpec((1, DH), lambda i: (0, 0)),
    ]
    if mode == 0:
        in_specs.append(pl.BlockSpec((K, DH), lambda i: (0, 0)))
    out_specs = [pl.BlockSpec((BR, DH), lambda i: (i, 0))]
    out_shape = [jax.ShapeDtypeStruct((N, DH), jnp.float32)]
    if mode < 2:
        out_specs.append(pl.BlockSpec((DH // LANE, BR, 64), lambda i: (0, i, 0)))
        out_shape.append(jax.ShapeDtypeStruct((DH // LANE, N, 64), jnp.int32))
    return pl.pallas_call(
        functools.partial(_layer_body, mode, C_in),
        grid=(N // BR,),
        in_specs=in_specs,
        out_specs=out_specs,
        out_shape=out_shape,
    )


def kernel(x, edge_index, l0_W1, l0_b1, l0_W2, l0_b2, l0_eps,
           l1_W1, l1_b1, l1_W2, l1_b2, l1_eps,
           l2_W1, l2_b1, l2_W2, l2_b2, l2_eps, l0_res_W):
    E = edge_index.shape[1]
    pad = E_PAD - E
    pad_edges = jnp.zeros((2, pad), jnp.int32)
    edges3 = jnp.concatenate([edge_index, pad_edges], axis=1).reshape(
        2, E_PAD // LANE, LANE
    )
    zeros = jnp.zeros((ROWS_PT, LANE), jnp.float32)
    ones = jnp.ones((LANE, LANE), jnp.float32)

    deg = _get_deg_kernel()(edges3, zeros, ones)

    x_c = _chunkify(x)
    S0 = _make_segsum(DIN // LANE)(x_c, edges3, zeros)
    h0, h0_c = _make_layer(DIN, DIN // LANE, 0)(
        x, S0, deg, (1.0 + l0_eps).reshape(1, 1), l0_W1, l0_b1.reshape(1, DH),
        l0_W2, l0_b2.reshape(1, DH), l0_res_W
    )
    S1 = _make_segsum(DH // LANE)(h0_c, edges3, zeros)
    h1, h1_c = _make_layer(DH, DH // LANE, 1)(
        h0, S1, deg, (1.0 + l1_eps).reshape(1, 1), l1_W1, l1_b1.reshape(1, DH),
        l1_W2, l1_b2.reshape(1, DH)
    )
    S2 = _make_segsum(DH // LANE)(h1_c, edges3, zeros)
    (h2,) = _make_layer(DH, DH // LANE, 2)(
        h1, S2, deg, (1.0 + l2_eps).reshape(1, 1), l2_W1, l2_b1.reshape(1, DH),
        l2_W2, l2_b2.reshape(1, DH)
    )
    return h2


# speedup vs baseline: 1.0053x; 1.0053x over previous
import functools

import jax
import jax.numpy as jnp
from jax import lax
from jax.experimental import pallas as pl
from jax.experimental.pallas import tpu as pltpu
from jax.experimental.pallas import tpu_sc as plsc

N = 10000
DIN = 256
DH = 512
LANE = 128
NC, NS = 2, 16
R = N
ROWS_PT = 640
ROWS_LAST = R - (NS - 1) * ROWS_PT
E_PAD = 163840
PAD_CNT = E_PAD - 160000


def _sc_mesh():
    return plsc.VectorSubcoreMesh(
        core_axis_name="c", subcore_axis_name="s", num_cores=NC, num_subcores=NS
    )


def _make_segsum(C):
    cps = C // NC
    nj = E_PAD // NS // LANE
    nh = nj // 2
    q_depth = 2

    def body(hc, edges, zeros, out, src_v, dst_v, buf, fbuf, acc, gsem):
        cid = lax.axis_index("c")
        sid = lax.axis_index("s")
        for k in range(cps):
            chunk = cid * cps + k

            @pl.when(sid < NS - 1)
            def _():
                pltpu.sync_copy(zeros, acc.at[pl.ds(sid * ROWS_PT, ROWS_PT)])

            @pl.when(sid == NS - 1)
            def _():
                pltpu.sync_copy(zeros.at[pl.ds(0, ROWS_LAST)],
                                acc.at[pl.ds(sid * ROWS_PT, ROWS_LAST)])

            plsc.subcore_barrier()
            for half in range(2):
                base = sid * nj + half * nh
                pltpu.sync_copy(edges.at[0, pl.ds(base, nh)], src_v)
                pltpu.sync_copy(edges.at[1, pl.ds(base, nh)], dst_v)
                for q in range(q_depth):
                    pltpu.async_copy(hc.at[chunk].at[src_v.at[q]], buf.at[q],
                                     gsem.at[q])

                def outer(jo, carry):
                    for q in range(q_depth):
                        j = jo * q_depth + q
                        pltpu.make_async_copy(
                            hc.at[chunk].at[src_v.at[q]], buf.at[q], gsem.at[q]
                        ).wait()

                        @plsc.parallel_loop(0, LANE, unroll=4)
                        def _(r):
                            for g in range(4):
                                v = buf[q, r, pl.ds(16 * g, 16)]
                                vb = plsc.bitcast(v, jnp.bfloat16)
                                a, b = plsc.unpack(
                                    vb, format=plsc.PackFormat.INTERLEAVED)
                                fbuf[r, pl.ds(16 * g, 16)] = a
                                fbuf[r, pl.ds(64 + 16 * g, 16)] = b

                        @pl.when(jo < nh // q_depth - 1)
                        def _():
                            pltpu.async_copy(
                                hc.at[chunk].at[src_v.at[j + q_depth]],
                                buf.at[q], gsem.at[q],
                            )
                        pltpu.sync_copy(fbuf, acc.at[dst_v.at[j]], add=True)
                    return carry

                lax.fori_loop(0, nh // q_depth, outer, 0)
            plsc.subcore_barrier()

            @pl.when(sid < NS - 1)
            def _():
                pltpu.sync_copy(
                    acc.at[pl.ds(sid * ROWS_PT, ROWS_PT)],
                    out.at[chunk, pl.ds(sid * ROWS_PT, ROWS_PT)],
                )

            @pl.when(sid == NS - 1)
            def _():
                pltpu.sync_copy(
                    acc.at[pl.ds(sid * ROWS_PT, ROWS_LAST)],
                    out.at[chunk, pl.ds(sid * ROWS_PT, ROWS_LAST)],
                )

            plsc.subcore_barrier()

    return pl.kernel(
        body,
        out_type=jax.ShapeDtypeStruct((C, R, LANE), jnp.float32),
        mesh=_sc_mesh(),
        compiler_params=pltpu.CompilerParams(
            needs_layout_passes=False, use_tc_tiling_on_sc=False),
        scratch_types=[
            pltpu.VMEM((nh, LANE), jnp.int32),
            pltpu.VMEM((nh, LANE), jnp.int32),
            pltpu.VMEM((q_depth, LANE, 64), jnp.int32),
            pltpu.VMEM((LANE, LANE), jnp.float32),
            pltpu.VMEM_SHARED((R, LANE), jnp.float32),
            pltpu.SemaphoreType.DMA((q_depth,)),
        ],
    )


def _deg_body(edges, zeros, ones, out, dst_v, buf, acc):
    njd = E_PAD // (NC * NS) // LANE
    cid = lax.axis_index("c")
    sid = lax.axis_index("s")
    wid = cid * NS + sid
    pltpu.sync_copy(edges.at[1, pl.ds(wid * njd, njd)], dst_v)
    pltpu.sync_copy(ones, buf)

    @pl.when(sid < NS - 1)
    def _():
        pltpu.sync_copy(zeros, acc.at[pl.ds(sid * ROWS_PT, ROWS_PT)])

    @pl.when(sid == NS - 1)
    def _():
        pltpu.sync_copy(zeros.at[pl.ds(0, ROWS_LAST)],
                        acc.at[pl.ds(sid * ROWS_PT, ROWS_LAST)])

    plsc.subcore_barrier()

    def step(j, carry):
        pltpu.sync_copy(buf, acc.at[dst_v.at[j]], add=True)
        return carry

    lax.fori_loop(0, njd, step, 0)
    plsc.subcore_barrier()

    @pl.when(sid < NS - 1)
    def _():
        pltpu.sync_copy(
            acc.at[pl.ds(sid * ROWS_PT, ROWS_PT)],
            out.at[cid, pl.ds(sid * ROWS_PT, ROWS_PT)],
        )

    @pl.when(sid == NS - 1)
    def _():
        pltpu.sync_copy(
            acc.at[pl.ds(sid * ROWS_PT, ROWS_LAST)],
            out.at[cid, pl.ds(sid * ROWS_PT, ROWS_LAST)],
        )


_deg_kernel = None


def _get_deg_kernel():
    global _deg_kernel
    if _deg_kernel is None:
        _deg_kernel = pl.kernel(
            _deg_body,
            out_type=jax.ShapeDtypeStruct((NC, R, LANE), jnp.float32),
            mesh=_sc_mesh(),
            scratch_types=[
                pltpu.VMEM((E_PAD // (NC * NS) // LANE, LANE), jnp.int32),
                pltpu.VMEM((LANE, LANE), jnp.float32),
                pltpu.VMEM_SHARED((R, LANE), jnp.float32),
            ],
        )
    return _deg_kernel


def _pack_chunk(X):
    lo = lax.bitcast_convert_type(
        X[:, :64].astype(jnp.bfloat16), jnp.uint16).astype(jnp.uint32)
    hi = lax.bitcast_convert_type(
        X[:, 64:].astype(jnp.bfloat16), jnp.uint16).astype(jnp.uint32)
    return lax.bitcast_convert_type(lo | (hi << 16), jnp.int32)


def _chunkify_body(x_ref, out_ref):
    for c in range(DIN // LANE):
        out_ref[c] = _pack_chunk(x_ref[:, c * LANE:(c + 1) * LANE])


def _chunkify(x):
    c = DIN // LANE
    return pl.pallas_call(
        _chunkify_body,
        grid=(N // BR,),
        in_specs=[pl.BlockSpec((BR, DIN), lambda i: (i, 0))],
        out_specs=pl.BlockSpec((c, BR, 64), lambda i: (0, i, 0)),
        out_shape=jax.ShapeDtypeStruct((c, N, 64), jnp.int32),
    )(x)


BR = 400


def _layer_body(mode, C_in, h_ref, S_ref, deg_ref, sc_ref, W1_ref, b1_ref,
                W2_ref, b2_ref, *rest):
    if mode == 0:
        Wres_ref, out_ref, hc_ref = rest
    elif mode == 1:
        out_ref, hc_ref = rest
    else:
        (out_ref,) = rest
    h = h_ref[...]
    corr = jnp.where(
        (lax.broadcasted_iota(jnp.int32, (h.shape[0], 1), 0) == 0)
        & (pl.program_id(0) == 0),
        jnp.float32(PAD_CNT), jnp.float32(0.0),
    )
    deg = jnp.maximum(deg_ref[0][:, :1] + deg_ref[1][:, :1] - corr, 1.0)
    S = jnp.concatenate([S_ref[c] for c in range(C_in)], axis=1)
    pre = sc_ref[...] * h + (S - corr * h) / deg
    mid = jnp.maximum(
        jnp.dot(pre, W1_ref[...], preferred_element_type=jnp.float32) + b1_ref[...],
        0.0,
    )
    out = jnp.dot(mid, W2_ref[...], preferred_element_type=jnp.float32) + b2_ref[...]
    if mode == 0:
        out = jnp.maximum(out, 0.0) + jnp.dot(
            h, Wres_ref[...], preferred_element_type=jnp.float32
        )
    elif mode == 1:
        out = jnp.maximum(out, 0.0) + h
    out_ref[...] = out
    if mode < 2:
        for c in range(DH // LANE):
            hc_ref[c] = _pack_chunk(out[:, c * LANE:(c + 1) * LANE])


def _make_layer(K, C_in, mode):
    in_specs = [
        pl.BlockSpec((BR, K), lambda i: (i, 0)),
        pl.BlockSpec((C_in, BR, LANE), lambda i: (0, i, 0)),
        pl.BlockSpec((NC, BR, LANE), lambda i: (0, i, 0)),
        pl.BlockSpec((1, 1), lambda i: (0, 0)),
        pl.BlockSpec((K, DH), lambda i: (0, 0)),
        pl.BlockSpec((1, DH), lambda i: (0, 0)),
        pl.BlockSpec((DH, DH), lambda i: (0, 0)),
        pl.BlockSpec((1, DH), lambda i: (0, 0)),
    ]
    if mode == 0:
        in_specs.append(pl.BlockSpec((K, DH), lambda i: (0, 0)))
    out_specs = [pl.BlockSpec((BR, DH), lambda i: (i, 0))]
    out_shape = [jax.ShapeDtypeStruct((N, DH), jnp.float32)]
    if mode < 2:
        out_specs.append(pl.BlockSpec((DH // LANE, BR, 64), lambda i: (0, i, 0)))
        out_shape.append(jax.ShapeDtypeStruct((DH // LANE, N, 64), jnp.int32))
    return pl.pallas_call(
        functools.partial(_layer_body, mode, C_in),
        grid=(N // BR,),
        in_specs=in_specs,
        out_specs=out_specs,
        out_shape=out_shape,
    )


def kernel(x, edge_index, l0_W1, l0_b1, l0_W2, l0_b2, l0_eps,
           l1_W1, l1_b1, l1_W2, l1_b2, l1_eps,
           l2_W1, l2_b1, l2_W2, l2_b2, l2_eps, l0_res_W):
    E = edge_index.shape[1]
    pad = E_PAD - E
    pad_edges = jnp.zeros((2, pad), jnp.int32)
    edges3 = jnp.concatenate([edge_index, pad_edges], axis=1).reshape(
        2, E_PAD // LANE, LANE
    )
    zeros = jnp.zeros((ROWS_PT, LANE), jnp.float32)
    ones = jnp.ones((LANE, LANE), jnp.float32)

    deg = _get_deg_kernel()(edges3, zeros, ones)

    x_c = _chunkify(x)
    S0 = _make_segsum(DIN // LANE)(x_c, edges3, zeros)
    h0, h0_c = _make_layer(DIN, DIN // LANE, 0)(
        x, S0, deg, (1.0 + l0_eps).reshape(1, 1), l0_W1, l0_b1.reshape(1, DH),
        l0_W2, l0_b2.reshape(1, DH), l0_res_W
    )
    S1 = _make_segsum(DH // LANE)(h0_c, edges3, zeros)
    h1, h1_c = _make_layer(DH, DH // LANE, 1)(
        h0, S1, deg, (1.0 + l1_eps).reshape(1, 1), l1_W1, l1_b1.reshape(1, DH),
        l1_W2, l1_b2.reshape(1, DH)
    )
    S2 = _make_segsum(DH // LANE)(h1_c, edges3, zeros)
    (h2,) = _make_layer(DH, DH // LANE, 2)(
        h1, S2, deg, (1.0 + l2_eps).reshape(1, 1), l2_W1, l2_b1.reshape(1, DH),
        l2_W2, l2_b2.reshape(1, DH)
    )
    return h2
